# baseline (speedup 1.0000x reference)
import jax
import jax.numpy as jnp
from jax import lax
from jax.experimental import pallas as pl
from jax.experimental.pallas import tpu as pltpu
from jax.experimental.pallas import tpu_sc as plsc

_VOCAB = 100000
_CTX = 2048
_D = 768
_BATCH = 4
_SEQ = 2048

_NC = 2
_NS = 16
_NW = _NC * _NS
_N = _BATCH * _SEQ
_P = _SEQ // _NW
_C = 32
_H = _P // _C
_LANES = 16


def _body(src_hbm, pos_hbm, emb_hbm, out_hbm,
          idx_v, pos_v, rows0, rows1,
          psem, gsem0, gsem1, osem0, osem1):
    wid = lax.axis_index("s") * _NC + lax.axis_index("c")
    pbase = wid * _P

    rows_bufs = [rows0, rows1]
    gsems = [gsem0, gsem1]
    osems = [osem0, osem1]

    pltpu.async_copy(pos_hbm.at[pl.ds(pbase, _P)], pos_v, psem)
    pltpu.sync_copy(src_hbm.at[wid], idx_v)

    def out_slice(b, h):
        return out_hbm.at[pl.ds(b * _SEQ + pbase + h * _C, _C)]

    def issue_gather(k):
        b, h = k // _H, k % _H
        pltpu.async_copy(emb_hbm.at[idx_v.at[b, pl.ds(h * _C, _C)]],
                         rows_bufs[h], gsems[h])

    issue_gather(0)
    pltpu.make_async_copy(pos_hbm.at[pl.ds(pbase, _P)], pos_v, psem).wait()

    for k in range(_BATCH * _H):
        b, h = k // _H, k % _H
        rows = rows_bufs[h]
        pltpu.make_async_copy(emb_hbm.at[idx_v.at[b, pl.ds(h * _C, _C)]],
                              rows, gsems[h]).wait()
        if k + 1 < _BATCH * _H:
            if k >= 1:
                pb, ph = (k - 1) // _H, (k - 1) % _H
                pltpu.make_async_copy(rows_bufs[ph], out_slice(pb, ph),
                                      osems[ph]).wait()
            issue_gather(k + 1)

        def row_body(r, carry):
            for j in range(_D // _LANES):
                s = pl.ds(j * _LANES, _LANES)
                rows[r, s] = rows[r, s] + pos_v[h * _C + r, s]
            return carry

        lax.fori_loop(0, _C, row_body, 0)

        pltpu.async_copy(rows, out_slice(b, h), osems[h])

    pltpu.make_async_copy(rows_bufs[0], out_slice(_BATCH - 1, 0),
                          osems[0]).wait()
    pltpu.make_async_copy(rows_bufs[1], out_slice(_BATCH - 1, 1),
                          osems[1]).wait()


@jax.jit
def _embed(src_t, emb_table, pos_table):
    kfn = pl.kernel(
        _body,
        out_type=jax.ShapeDtypeStruct((_N, _D), jnp.float32),
        mesh=plsc.VectorSubcoreMesh(core_axis_name="c", subcore_axis_name="s",
                                    num_cores=_NC, num_subcores=_NS),
        scratch_types=[
            pltpu.VMEM((_BATCH, _P), jnp.int32),
            pltpu.VMEM((_P, _D), jnp.float32),
            pltpu.VMEM((_C, _D), jnp.float32),
            pltpu.VMEM((_C, _D), jnp.float32),
            pltpu.SemaphoreType.DMA,
            pltpu.SemaphoreType.DMA,
            pltpu.SemaphoreType.DMA,
            pltpu.SemaphoreType.DMA,
            pltpu.SemaphoreType.DMA,
        ],
    )
    return kfn(src_t, pos_table, emb_table)


def kernel(src, emb_table, pos_table):
    batch, seq = src.shape
    src_t = src.reshape(batch, _NW, _P).transpose(1, 0, 2).astype(jnp.int32)
    out = _embed(src_t, emb_table, pos_table)
    return out.reshape(batch, seq, _D)


# speedup vs baseline: 1.1076x; 1.1076x over previous
import jax
import jax.numpy as jnp
from jax import lax
from jax.experimental import pallas as pl
from jax.experimental.pallas import tpu as pltpu
from jax.experimental.pallas import tpu_sc as plsc

_VOCAB = 100000
_CTX = 2048
_D = 768
_BATCH = 4
_SEQ = 2048

_NC = 2
_NS = 16
_NW = _NC * _NS
_N = _BATCH * _SEQ
_P = _SEQ // _NW
_C = 32
_H = _P // _C
_LANES = 16


def _body(src_hbm, pos_hbm, emb_hbm, out_hbm,
          idx_v, pos_v, rows0, rows1,
          psem, gsem0, gsem1, osem0, osem1):
    wid = lax.axis_index("s") * _NC + lax.axis_index("c")
    pbase = wid * _P

    rows_bufs = [rows0, rows1]
    gsems = [gsem0, gsem1]
    osems = [osem0, osem1]

    pltpu.async_copy(pos_hbm.at[pl.ds(pbase, _P)], pos_v, psem)
    pltpu.sync_copy(src_hbm.at[wid], idx_v)

    def out_slice(b, h):
        return out_hbm.at[pl.ds(b * _SEQ + pbase + h * _C, _C)]

    def issue_gather(k):
        h = k % _H
        pltpu.async_copy(emb_hbm.at[idx_v.at[k]], rows_bufs[h], gsems[h])

    issue_gather(0)
    pltpu.make_async_copy(pos_hbm.at[pl.ds(pbase, _P)], pos_v, psem).wait()

    for k in range(_BATCH * _H):
        b, h = k // _H, k % _H
        rows = rows_bufs[h]
        pltpu.make_async_copy(emb_hbm.at[idx_v.at[k]], rows, gsems[h]).wait()
        if k + 1 < _BATCH * _H:
            if k >= 1:
                pb, ph = (k - 1) // _H, (k - 1) % _H
                pltpu.make_async_copy(rows_bufs[ph], out_slice(pb, ph),
                                      osems[ph]).wait()
            issue_gather(k + 1)

        def row_body(r, carry):
            for j in range(_D // _LANES):
                s = pl.ds(j * _LANES, _LANES)
                plsc.addupdate(rows.at[r, s], pos_v[h * _C + r, s])
            return carry

        lax.fori_loop(0, _C, row_body, 0)

        pltpu.async_copy(rows, out_slice(b, h), osems[h])

    pltpu.make_async_copy(rows_bufs[0], out_slice(_BATCH - 1, 0),
                          osems[0]).wait()
    pltpu.make_async_copy(rows_bufs[1], out_slice(_BATCH - 1, 1),
                          osems[1]).wait()


@jax.jit
def _embed(src_t, emb_table, pos_table):
    kfn = pl.kernel(
        _body,
        out_type=jax.ShapeDtypeStruct((_N, _D), jnp.float32),
        mesh=plsc.VectorSubcoreMesh(core_axis_name="c", subcore_axis_name="s",
                                    num_cores=_NC, num_subcores=_NS),
        scratch_types=[
            pltpu.VMEM((_BATCH * _H, _C), jnp.int32),
            pltpu.VMEM((_P, _D), jnp.float32),
            pltpu.VMEM((_C, _D), jnp.float32),
            pltpu.VMEM((_C, _D), jnp.float32),
            pltpu.SemaphoreType.DMA,
            pltpu.SemaphoreType.DMA,
            pltpu.SemaphoreType.DMA,
            pltpu.SemaphoreType.DMA,
            pltpu.SemaphoreType.DMA,
        ],
    )
    return kfn(src_t, pos_table, emb_table)


def kernel(src, emb_table, pos_table):
    batch, seq = src.shape
    src_t = (src.reshape(batch, _NW, _H, _C).transpose(1, 0, 2, 3)
             .reshape(_NW, batch * _H, _C).astype(jnp.int32))
    out = _embed(src_t, emb_table, pos_table)
    return out.reshape(batch, seq, _D)


# speedup vs baseline: 1.1562x; 1.0439x over previous
import jax
import jax.numpy as jnp
from jax import lax
from jax.experimental import pallas as pl
from jax.experimental.pallas import tpu as pltpu
from jax.experimental.pallas import tpu_sc as plsc

_VOCAB = 100000
_CTX = 2048
_D = 768
_BATCH = 4
_SEQ = 2048

_NC = 2
_NS = 16
_NW = _NC * _NS
_N = _BATCH * _SEQ
_P = _SEQ // _NW
_C = 32
_H = _P // _C
_NCHUNK = _BATCH * _H
_NBUF = 3
_LANES = 16


def _body(src_hbm, pos_hbm, emb_hbm, out_hbm,
          idx_v, pos_v, rows0, rows1, rows2,
          psem, gsem0, gsem1, gsem2, osem0, osem1, osem2):
    wid = lax.axis_index("s") * _NC + lax.axis_index("c")
    pbase = wid * _P

    rows_bufs = [rows0, rows1, rows2]
    gsems = [gsem0, gsem1, gsem2]
    osems = [osem0, osem1, osem2]

    pltpu.async_copy(pos_hbm.at[pl.ds(pbase, _P)], pos_v, psem)
    pltpu.sync_copy(src_hbm.at[wid], idx_v)

    def out_slice(k):
        b, h = k // _H, k % _H
        return out_hbm.at[pl.ds(b * _SEQ + pbase + h * _C, _C)]

    def issue_gather(k):
        nb = k % _NBUF
        pltpu.async_copy(emb_hbm.at[idx_v.at[k]], rows_bufs[nb], gsems[nb])

    issue_gather(0)
    issue_gather(1)
    pltpu.make_async_copy(pos_hbm.at[pl.ds(pbase, _P)], pos_v, psem).wait()

    for k in range(_NCHUNK):
        nb = k % _NBUF
        h = k % _H
        rows = rows_bufs[nb]
        pltpu.make_async_copy(emb_hbm.at[idx_v.at[k]], rows, gsems[nb]).wait()
        if k + 2 < _NCHUNK:
            if k >= 1:
                pnb = (k - 1) % _NBUF
                pltpu.make_async_copy(rows_bufs[pnb], out_slice(k - 1),
                                      osems[pnb]).wait()
            issue_gather(k + 2)

        def row_body(r, carry):
            for j in range(_D // _LANES):
                s = pl.ds(j * _LANES, _LANES)
                plsc.addupdate(rows.at[r, s], pos_v[h * _C + r, s])
            return carry

        lax.fori_loop(0, _C, row_body, 0)

        pltpu.async_copy(rows, out_slice(k), osems[nb])

    for k in range(_NCHUNK - _NBUF, _NCHUNK):
        nb = k % _NBUF
        pltpu.make_async_copy(rows_bufs[nb], out_slice(k), osems[nb]).wait()


@jax.jit
def _embed(src_t, emb_table, pos_table):
    kfn = pl.kernel(
        _body,
        out_type=jax.ShapeDtypeStruct((_N, _D), jnp.float32),
        mesh=plsc.VectorSubcoreMesh(core_axis_name="c", subcore_axis_name="s",
                                    num_cores=_NC, num_subcores=_NS),
        scratch_types=[
            pltpu.VMEM((_NCHUNK, _C), jnp.int32),
            pltpu.VMEM((_P, _D), jnp.float32),
            pltpu.VMEM((_C, _D), jnp.float32),
            pltpu.VMEM((_C, _D), jnp.float32),
            pltpu.VMEM((_C, _D), jnp.float32),
            pltpu.SemaphoreType.DMA,
            pltpu.SemaphoreType.DMA,
            pltpu.SemaphoreType.DMA,
            pltpu.SemaphoreType.DMA,
            pltpu.SemaphoreType.DMA,
            pltpu.SemaphoreType.DMA,
            pltpu.SemaphoreType.DMA,
        ],
    )
    return kfn(src_t, pos_table, emb_table)


def kernel(src, emb_table, pos_table):
    batch, seq = src.shape
    src_t = (src.reshape(batch, _NW, _H, _C).transpose(1, 0, 2, 3)
             .reshape(_NW, _NCHUNK, _C).astype(jnp.int32))
    out = _embed(src_t, emb_table, pos_table)
    return out.reshape(batch, seq, _D)


# speedup vs baseline: 1.2682x; 1.0969x over previous
import jax
import jax.numpy as jnp
from jax import lax
from jax.experimental import pallas as pl
from jax.experimental.pallas import tpu as pltpu
from jax.experimental.pallas import tpu_sc as plsc

_VOCAB = 100000
_CTX = 2048
_D = 768
_BATCH = 4
_SEQ = 2048

_NC = 2
_NS = 16
_NW = _NC * _NS
_N = _BATCH * _SEQ
_PER_W = _N // _NW
_C = 32
_NCHUNK = _PER_W // _C
_LANES = 16


def _body(src_hbm, pos_hbm, emb_hbm, out_hbm,
          idx_v, rows0, rows1, pos0, pos1,
          gsem0, gsem1, psem0, psem1):
    wid = lax.axis_index("s") * _NC + lax.axis_index("c")
    base = wid * _PER_W
    pos_base = lax.rem(base, _SEQ)

    rows_bufs = [rows0, rows1]
    pos_bufs = [pos0, pos1]
    gsems = [gsem0, gsem1]
    psems = [psem0, psem1]

    pltpu.sync_copy(src_hbm.at[wid], idx_v)

    def issue(c):
        nb = c % 2
        pltpu.async_copy(emb_hbm.at[idx_v.at[c]], rows_bufs[nb], gsems[nb])
        pltpu.async_copy(pos_hbm.at[pl.ds(pos_base + c * _C, _C)],
                         pos_bufs[nb], psems[nb])

    issue(0)
    for c in range(_NCHUNK):
        nb = c % 2
        pltpu.make_async_copy(emb_hbm.at[idx_v.at[c]], rows_bufs[nb],
                              gsems[nb]).wait()
        pltpu.make_async_copy(pos_hbm.at[pl.ds(pos_base + c * _C, _C)],
                              pos_bufs[nb], psems[nb]).wait()
        if c + 1 < _NCHUNK:
            issue(c + 1)

        rows = rows_bufs[nb]
        pos = pos_bufs[nb]

        def row_body(r, carry):
            for j in range(_D // _LANES):
                s = pl.ds(j * _LANES, _LANES)
                plsc.addupdate(rows.at[r, s], pos[r, s])
            return carry

        lax.fori_loop(0, _C, row_body, 0)

        pltpu.sync_copy(rows, out_hbm.at[pl.ds(base + c * _C, _C)])


@jax.jit
def _embed(src_flat, emb_table, pos_table):
    kfn = pl.kernel(
        _body,
        out_type=jax.ShapeDtypeStruct((_N, _D), jnp.float32),
        mesh=plsc.VectorSubcoreMesh(core_axis_name="c", subcore_axis_name="s",
                                    num_cores=_NC, num_subcores=_NS),
        scratch_types=[
            pltpu.VMEM((_NCHUNK, _C), jnp.int32),
            pltpu.VMEM((_C, _D), jnp.float32),
            pltpu.VMEM((_C, _D), jnp.float32),
            pltpu.VMEM((_C, _D), jnp.float32),
            pltpu.VMEM((_C, _D), jnp.float32),
            pltpu.SemaphoreType.DMA,
            pltpu.SemaphoreType.DMA,
            pltpu.SemaphoreType.DMA,
            pltpu.SemaphoreType.DMA,
        ],
    )
    return kfn(src_flat, pos_table, emb_table)


def kernel(src, emb_table, pos_table):
    batch, seq = src.shape
    src_flat = src.reshape(_NW, _NCHUNK, _C).astype(jnp.int32)
    out = _embed(src_flat, emb_table, pos_table)
    return out.reshape(batch, seq, _D)
